# Initial kernel scaffold; baseline (speedup 1.0000x reference)
import jax
import jax.numpy as jnp
from jax.experimental import pallas as pl


def kernel(grid_size, pos_embed_table):
    raise NotImplementedError("write your pallas kernel here")


# speedup vs baseline: 1.6054x; 1.6054x over previous
import functools

import jax
import jax.numpy as jnp
from jax import lax
from jax.experimental import pallas as pl
from jax.experimental.pallas import tpu as pltpu
from jax.experimental.pallas import tpu_sc as plsc

B = 16384
D = 1024
MAXW = 128
NC = 2
NS = 16
NW = NC * NS
RPW = B // NW
CH = 32
NCH = RPW // CH
LANES = 16


@functools.partial(
    pl.kernel,
    out_type=jax.ShapeDtypeStruct((B, D), jnp.float32),
    mesh=plsc.VectorSubcoreMesh(core_axis_name="c", subcore_axis_name="s"),
    scratch_types=[
        pltpu.VMEM((CH,), jnp.int32),
        pltpu.VMEM((CH,), jnp.int32),
        pltpu.VMEM((LANES,), jnp.int32),
        pltpu.VMEM((LANES,), jnp.int32),
        pltpu.VMEM((CH, D), jnp.float32),
        pltpu.VMEM((CH, D), jnp.float32),
        pltpu.SemaphoreType.DMA,
        pltpu.SemaphoreType.DMA,
        pltpu.SemaphoreType.DMA,
        pltpu.SemaphoreType.DMA,
    ],
)
def _pos_gather(hmax_hbm, wmax_hbm, table_hbm, out_hbm,
                idx0, idx1, hv_v, wv_v, buf0, buf1,
                g0, g1, o0, o1):
    wid = lax.axis_index("s") * NC + lax.axis_index("c")
    base = wid * RPW

    pltpu.sync_copy(hmax_hbm, hv_v)
    pltpu.sync_copy(wmax_hbm, wv_v)
    hmax = hv_v[...]
    wmax = wv_v[...]

    lane = lax.iota(jnp.int32, LANES)

    def compute_idx(c, dst):
        for i in range(CH // LANES):
            p = lane + (base + c * CH + i * LANES)
            row = lax.shift_right_logical(p, 7)
            col = lax.bitwise_and(p, MAXW - 1)
            valid = (row < hmax) & (col < wmax)
            dst[pl.ds(i * LANES, LANES)] = jnp.where(valid, p, 0)

    bufs = (buf0, buf1)
    idxs = (idx0, idx1)
    gsems = (g0, g1)
    osems = (o0, o1)
    gathers = [None, None]
    out_pending = [None, None]

    def start_gather(c):
        b = c & 1
        compute_idx(c, idxs[b])
        gathers[b] = pltpu.async_copy(
            table_hbm.at[idxs[b]], bufs[b], gsems[b])

    start_gather(0)
    for c in range(NCH):
        b = c & 1
        gathers[b].wait()
        if c + 1 < NCH:
            b2 = (c + 1) & 1
            if out_pending[b2] is not None:
                out_pending[b2].wait()
                out_pending[b2] = None
            start_gather(c + 1)
        out_pending[b] = pltpu.async_copy(
            bufs[b], out_hbm.at[pl.ds(base + c * CH, CH)], osems[b])
    for b in range(2):
        if out_pending[b] is not None:
            out_pending[b].wait()


def kernel(grid_size, pos_embed_table):
    table = pos_embed_table.reshape(B, D)
    hmax = jnp.broadcast_to(grid_size[0].astype(jnp.int32), (LANES,))
    wmax = jnp.broadcast_to(grid_size[1].astype(jnp.int32), (LANES,))
    out = _pos_gather(hmax, wmax, table)
    return out.reshape(1, B, D)


# speedup vs baseline: 1.6161x; 1.0066x over previous
import functools

import jax
import jax.numpy as jnp
from jax import lax
from jax.experimental import pallas as pl
from jax.experimental.pallas import tpu as pltpu
from jax.experimental.pallas import tpu_sc as plsc

B = 16384
D = 1024
MAXW = 128
NC = 2
NS = 16
NW = NC * NS
RPW = B // NW
CH = 32
NCH = RPW // CH
NBUF = 3
LANES = 16


@functools.partial(
    pl.kernel,
    out_type=jax.ShapeDtypeStruct((B, D), jnp.float32),
    mesh=plsc.VectorSubcoreMesh(core_axis_name="c", subcore_axis_name="s"),
    scratch_types=(
        [pltpu.VMEM((CH,), jnp.int32) for _ in range(NBUF)]
        + [pltpu.VMEM((LANES,), jnp.int32),
           pltpu.VMEM((LANES,), jnp.int32)]
        + [pltpu.VMEM((CH, D), jnp.float32) for _ in range(NBUF)]
        + [pltpu.SemaphoreType.DMA for _ in range(2 * NBUF)]
    ),
)
def _pos_gather(hmax_hbm, wmax_hbm, table_hbm, out_hbm,
                idx0, idx1, idx2, hv_v, wv_v, buf0, buf1, buf2,
                g0, g1, g2, o0, o1, o2):
    wid = lax.axis_index("s") * NC + lax.axis_index("c")
    base = wid * RPW

    pltpu.sync_copy(hmax_hbm, hv_v)
    pltpu.sync_copy(wmax_hbm, wv_v)
    hmax = hv_v[...]
    wmax = wv_v[...]

    lane = lax.iota(jnp.int32, LANES)

    def compute_idx(c, dst):
        for i in range(CH // LANES):
            p = lane + (base + c * CH + i * LANES)
            row = lax.shift_right_logical(p, 7)
            col = lax.bitwise_and(p, MAXW - 1)
            valid = (row < hmax) & (col < wmax)
            dst[pl.ds(i * LANES, LANES)] = jnp.where(valid, p, 0)

    bufs = (buf0, buf1, buf2)
    idxs = (idx0, idx1, idx2)
    gsems = (g0, g1, g2)
    osems = (o0, o1, o2)
    gathers = [None] * NBUF
    out_pending = [None] * NBUF

    def start_gather(c):
        b = c % NBUF
        compute_idx(c, idxs[b])
        gathers[b] = pltpu.async_copy(
            table_hbm.at[idxs[b]], bufs[b], gsems[b])

    for c in range(NBUF - 1):
        start_gather(c)
    for c in range(NCH):
        b = c % NBUF
        gathers[b].wait()
        out_pending[b] = pltpu.async_copy(
            bufs[b], out_hbm.at[pl.ds(base + c * CH, CH)], osems[b])
        n = c + NBUF - 1
        if n < NCH:
            bn = n % NBUF
            if out_pending[bn] is not None:
                out_pending[bn].wait()
                out_pending[bn] = None
            start_gather(n)
    for b in range(NBUF):
        if out_pending[b] is not None:
            out_pending[b].wait()


def kernel(grid_size, pos_embed_table):
    table = pos_embed_table.reshape(B, D)
    hmax = jnp.broadcast_to(grid_size[0].astype(jnp.int32), (LANES,))
    wmax = jnp.broadcast_to(grid_size[1].astype(jnp.int32), (LANES,))
    out = _pos_gather(hmax, wmax, table)
    return out.reshape(1, B, D)
